# baseline (speedup 1.0000x reference)
import functools

import jax
import jax.numpy as jnp
from jax import lax
from jax.experimental import pallas as pl
from jax.experimental.pallas import tpu as pltpu
from jax.experimental.pallas import tpu_sc as plsc

N = 10000
E = 160000
NCORES = 2
NSUB = 16
NW = NCORES * NSUB
EPW = E // NW
ET = E // NSUB
EC = 125
NCH = ET // EC
DC = 100
NDC = EPW // DC
RB = 624
ZR = 48
P = 128

_mesh = plsc.VectorSubcoreMesh(core_axis_name="c", subcore_axis_name="s")
_sc_params = pltpu.CompilerParams(needs_layout_passes=False)


NDC2 = ET // DC


def _nrsqrt(x):
    xi = plsc.bitcast(x, jnp.int32)
    y = plsc.bitcast(jnp.full((16,), 0x5F3759DF, jnp.int32)
                     - lax.shift_right_logical(xi, 1), jnp.float32)
    hx = x * (-0.5)
    for _ in range(3):
        y = y * (hx * y * y + 1.5)
    return y


@functools.partial(
    pl.kernel, mesh=_mesh, compiler_params=_sc_params,
    out_type=[jax.ShapeDtypeStruct((NW, 1, EPW), jnp.float32),
              jax.ShapeDtypeStruct((1, N), jnp.float32)],
    scratch_types=[
        pltpu.VMEM((NDC2, DC), jnp.int32),
        pltpu.VMEM((NDC2, DC), jnp.float32),
        pltpu.VMEM((N,), jnp.float32),
        pltpu.VMEM((704,), jnp.float32),
        pltpu.VMEM((EPW + 16,), jnp.int32),
        pltpu.VMEM((EPW + 16,), jnp.int32),
        pltpu.VMEM((EPW + 16,), jnp.float32),
        pltpu.VMEM((EPW + 16,), jnp.float32),
        pltpu.VMEM((N,), jnp.float32),
        pltpu.VMEM_SHARED((N,), jnp.float32),
        pltpu.VMEM_SHARED((N,), jnp.float32),
        pltpu.SemaphoreType.DMA,
    ],
)
def _prep_kernel(dstd_hbm, wd_hbm, src_hbm, dst_hbm, w_hbm,
                 norm_hbm, dinv2_hbm,
                 dstv, wdv, dinvv, dslice, sv, dv, wv, nv, d2buf,
                 dacc, dtab, sem):
    c = lax.axis_index("c")
    s = lax.axis_index("s")
    wid = c * NSUB + s
    rows = jnp.where(s == NSUB - 1, RB + 16, RB)
    base = s * RB

    def z(i, carry):
        dinvv[pl.ds(i * 16, 16)] = jnp.zeros((16,), jnp.float32)
        return carry

    lax.fori_loop(0, N // 16, z, 0)
    pltpu.sync_copy(dstd_hbm.at[s], dstv)
    pltpu.sync_copy(wd_hbm.at[s], wdv)

    @pl.when(s == 0)
    def _():
        pltpu.sync_copy(dinvv, dacc)

    plsc.subcore_barrier()

    def cb(j, carry):
        pltpu.sync_copy(wdv.at[j], dacc.at[dstv.at[j]], add=True)
        return carry

    lax.fori_loop(0, NDC2, cb, 0)
    plsc.subcore_barrier()

    pltpu.sync_copy(dacc.at[pl.ds(base, RB)], dslice.at[pl.ds(0, RB)])

    @pl.when(s == NSUB - 1)
    def _():
        pltpu.sync_copy(dacc.at[pl.ds(NSUB * RB, 16)],
                        dslice.at[pl.ds(RB, 16)])

    def nb_(i, carry):
        deg = dslice[pl.ds(i * 16, 16)] + 1.0
        dslice[pl.ds(i * 16, 16)] = _nrsqrt(deg)
        return carry

    lax.fori_loop(0, 704 // 16, nb_, 0)
    pltpu.sync_copy(dslice.at[pl.ds(0, RB)], dtab.at[pl.ds(base, RB)])

    @pl.when(s == NSUB - 1)
    def _():
        pltpu.sync_copy(dslice.at[pl.ds(RB, 16)],
                        dtab.at[pl.ds(NSUB * RB, 16)])

    plsc.subcore_barrier()
    pltpu.sync_copy(dtab, dinvv)

    @pl.when((c == 0) & (s == 0))
    def _():
        def d2(i, carry):
            di = dinvv[pl.ds(i * 16, 16)]
            d2buf[pl.ds(i * 16, 16)] = di * di
            return carry

        lax.fori_loop(0, N // 16, d2, 0)
        pltpu.sync_copy(d2buf, dinv2_hbm.at[0])
    pltpu.sync_copy(src_hbm.at[wid, 0], sv.at[pl.ds(0, EPW)])
    pltpu.sync_copy(dst_hbm.at[wid, 0], dv.at[pl.ds(0, EPW)])
    pltpu.sync_copy(w_hbm.at[wid, 0], wv.at[pl.ds(0, EPW)])

    nit = pl.cdiv(EPW, 16)

    def eb(j, carry):
        si = jnp.clip(sv[pl.ds(j * 16, 16)], 0, N - 1)
        di = jnp.clip(dv[pl.ds(j * 16, 16)], 0, N - 1)
        gs = plsc.load_gather(dinvv, [si])
        gd = plsc.load_gather(dinvv, [di])
        nv[pl.ds(j * 16, 16)] = gs * wv[pl.ds(j * 16, 16)] * gd
        return carry

    lax.fori_loop(0, nit, eb, 0)
    pltpu.sync_copy(nv.at[pl.ds(0, EPW)], norm_hbm.at[wid, 0])


NB = 3
EB = 3
UNR = 6


def _make_agg(npan):
    ppc = npan // NCORES

    @functools.partial(
        pl.kernel, mesh=_mesh, compiler_params=_sc_params,
        out_type=[jax.ShapeDtypeStruct((N, P), jnp.float32)] * npan,
        scratch_types=(
            [pltpu.VMEM((3, EC), jnp.int32) for _ in range(EB)] +
            [pltpu.VMEM((EC, P), jnp.float32) for _ in range(NB)] +
            [pltpu.VMEM_SHARED((N, P), jnp.float32)] +
            [pltpu.SemaphoreType.DMA for _ in range(2 * NB + EB)]
        ),
    )
    def agg(*refs):
        ys = refs[:npan]
        edata_hbm = refs[npan]
        outs = refs[npan + 1:npan + 1 + npan]
        scr = refs[npan + 1 + npan:]
        ebufs = scr[0:EB]
        rbufs = scr[EB:EB + NB]
        acc = scr[EB + NB]
        sgs = scr[EB + NB + 1:EB + NB + 1 + NB]
        sss = scr[EB + NB + 1 + NB:EB + NB + 1 + 2 * NB]
        ses = scr[EB + NB + 1 + 2 * NB:]

        c = lax.axis_index("c")
        s = lax.axis_index("s")

        def scale(rows, ebuf):
            two = jnp.full((16,), 2, jnp.int32)

            @plsc.parallel_loop(0, EC, 1, unroll=UNR)
            def _(e):
                nbi = plsc.load_gather(
                    ebuf, [two, jnp.full((16,), e, jnp.int32)])
                nb = plsc.bitcast(nbi, jnp.float32)
                for k in range(P // 16):
                    rows[e, pl.ds(k * 16, 16)] = (
                        rows[e, pl.ds(k * 16, 16)] * nb)

        def run_panel(yref, oref):
            z0 = rbufs[0]

            def zrows(i, carry):
                for k in range(P // 16):
                    z0[i, pl.ds(k * 16, 16)] = jnp.zeros((16,), jnp.float32)
                return carry

            lax.fori_loop(0, EC, zrows, 0)
            for k in range(RB // 96):
                pltpu.sync_copy(z0.at[pl.ds(0, 96)],
                                acc.at[pl.ds(s * RB + k * 96, 96)])
            pltpu.sync_copy(z0.at[pl.ds(0, 48)],
                            acc.at[pl.ds(s * RB + 576, 48)])

            @pl.when(s == NSUB - 1)
            def _():
                pltpu.sync_copy(z0.at[pl.ds(0, 16)],
                                acc.at[pl.ds(NSUB * RB, 16)])

            plsc.subcore_barrier()

            pltpu.sync_copy(edata_hbm.at[s, 0], ebufs[0])
            pltpu.async_copy(yref.at[ebufs[0].at[0]], rbufs[0], sgs[0])
            pltpu.async_copy(edata_hbm.at[s, 1], ebufs[1], ses[1])

            def maybe_when(cond, fn):
                if isinstance(cond, bool):
                    if cond:
                        fn()
                else:
                    pl.when(cond)(fn)

            def step(ch, b):
                nb_ = (b + 1) % NB
                pb = (b + NB - 1) % NB

                def _gather():
                    pltpu.make_async_copy(
                        edata_hbm.at[s, ch + 1], ebufs[nb_], ses[nb_]).wait()
                    pltpu.async_copy(yref.at[ebufs[nb_].at[0]],
                                     rbufs[nb_], sgs[nb_])

                maybe_when(ch + 1 < NCH, _gather)

                pltpu.make_async_copy(
                    yref.at[ebufs[b].at[0]], rbufs[b], sgs[b]).wait()
                scale(rbufs[b], ebufs[b])

                def _drain():
                    pltpu.make_async_copy(
                        rbufs[pb], acc.at[ebufs[pb].at[1]], sss[pb]).wait()

                maybe_when(ch >= 1, _drain)

                def _pref():
                    pltpu.async_copy(edata_hbm.at[s, ch + 2],
                                     ebufs[pb], ses[pb])

                maybe_when(ch + 2 < NCH, _pref)

                pltpu.async_copy(rbufs[b], acc.at[ebufs[b].at[1]], sss[b],
                                 add=True)

            NMAIN = (NCH // NB) * NB

            def pack_body(g, carry):
                for u in range(NB):
                    step(g * NB + u, u)
                return carry

            lax.fori_loop(0, NCH // NB, pack_body, 0)
            for ch in range(NMAIN, NCH):
                step(ch, ch % NB)
            pltpu.make_async_copy(
                rbufs[(NCH - 1) % NB],
                acc.at[ebufs[(NCH - 1) % NB].at[1]],
                sss[(NCH - 1) % NB]).wait()

            plsc.subcore_barrier()
            pltpu.sync_copy(acc.at[pl.ds(s * RB, RB)],
                            oref.at[pl.ds(s * RB, RB)])

            @pl.when(s == NSUB - 1)
            def _():
                pltpu.sync_copy(acc.at[pl.ds(NSUB * RB, 16)],
                                oref.at[pl.ds(NSUB * RB, 16)])

            plsc.subcore_barrier()

        for cid in range(NCORES):
            @pl.when(c == cid)
            def _(cid=cid):
                for p in range(ppc):
                    pan = cid * ppc + p
                    run_panel(ys[pan], outs[pan])

    return agg


_agg4 = _make_agg(4)
_agg2 = _make_agg(2)


_BM = 256


def _mm(x, W, npan):
    K = x.shape[1]
    F = W.shape[1]

    def body(x_ref, w_ref, *o_refs):
        y = jnp.dot(x_ref[...], w_ref[...],
                    preferred_element_type=jnp.float32)
        for i, o in enumerate(o_refs):
            o[...] = y[:, i * P:(i + 1) * P]

    return pl.pallas_call(
        body,
        grid=(pl.cdiv(N, _BM),),
        in_specs=[pl.BlockSpec((_BM, K), lambda i: (i, 0)),
                  pl.BlockSpec((K, F), lambda i: (0, 0))],
        out_specs=[pl.BlockSpec((_BM, P), lambda i: (i, 0))] * npan,
        out_shape=[jax.ShapeDtypeStruct((N, P), jnp.float32)] * npan,
    )(x, W)


def _fuse(aggs, xws, dinv2c, b2d, W=None, npan_out=0, emit_h=False):
    npan_in = len(aggs)
    F = npan_in * P
    have_w = W is not None

    def body(*refs):
        k = 0
        aggr = refs[k:k + npan_in]; k += npan_in
        xwr = refs[k:k + npan_in]; k += npan_in
        d2 = refs[k]; br = refs[k + 1]; k += 2
        wr = None
        if have_w:
            wr = refs[k]; k += 1
        outs = refs[k:]
        agg = jnp.concatenate([r[...] for r in aggr], axis=1)
        xw = jnp.concatenate([r[...] for r in xwr], axis=1)
        h = jnp.maximum(agg + d2[...] * xw + br[...], 0.0)
        oi = 0
        if emit_h:
            outs[oi][...] = h
            oi += 1
        if have_w:
            y = jnp.dot(h, wr[...], preferred_element_type=jnp.float32)
            for i in range(npan_out):
                outs[oi + i][...] = y[:, i * P:(i + 1) * P]

    in_specs = ([pl.BlockSpec((_BM, P), lambda i: (i, 0))] * (2 * npan_in) +
                [pl.BlockSpec((_BM, 1), lambda i: (i, 0)),
                 pl.BlockSpec((1, F), lambda i: (0, 0))])
    args = list(aggs) + list(xws) + [dinv2c, b2d]
    if have_w:
        in_specs.append(pl.BlockSpec(W.shape, lambda i: (0, 0)))
        args.append(W)
    out_shape, out_specs = [], []
    if emit_h:
        out_shape.append(jax.ShapeDtypeStruct((N, F), jnp.float32))
        out_specs.append(pl.BlockSpec((_BM, F), lambda i: (i, 0)))
    out_shape += [jax.ShapeDtypeStruct((N, P), jnp.float32)] * npan_out
    out_specs += [pl.BlockSpec((_BM, P), lambda i: (i, 0))] * npan_out

    res = pl.pallas_call(
        body,
        grid=(pl.cdiv(N, _BM),),
        in_specs=in_specs,
        out_specs=out_specs,
        out_shape=out_shape,
    )(*args)
    return res


def kernel(x, edge_index, edge_weight, W1, b1, W2, b2, W3, b3, W4, b4):
    src = edge_index[0]
    dst = edge_index[1]

    norm, dinv2_row = _prep_kernel(
        dst.reshape(NSUB, NDC2, DC), edge_weight.reshape(NSUB, NDC2, DC),
        src.reshape(NW, 1, EPW), dst.reshape(NW, 1, EPW),
        edge_weight.reshape(NW, 1, EPW))
    dinv2c = dinv2_row.reshape(N, 1)
    norm_i = lax.bitcast_convert_type(norm.reshape(NSUB, NCH, EC), jnp.int32)
    edata = jnp.stack([src.reshape(NSUB, NCH, EC),
                       dst.reshape(NSUB, NCH, EC), norm_i], axis=2)

    xw1 = _mm(x, W1, 4)
    agg1 = _agg4(*xw1, edata)
    xw2 = _fuse(agg1, xw1, dinv2c, b1.reshape(1, -1), W=W2, npan_out=2)
    agg2 = _agg2(*xw2, edata)
    xw3 = _fuse(agg2, xw2, dinv2c, b2.reshape(1, -1), W=W3, npan_out=4)
    agg3 = _agg4(*xw3, edata)
    res3 = _fuse(agg3, xw3, dinv2c, b3.reshape(1, -1), W=W4, npan_out=2,
                 emit_h=True)
    x_emb = res3[0]
    xw4 = res3[1:]
    agg4 = _agg2(*xw4, edata)
    out = _fuse(agg4, xw4, dinv2c, b4.reshape(1, -1), emit_h=True)[0]
    return (out, x_emb)


# speedup vs baseline: 1.0248x; 1.0248x over previous
import functools

import jax
import jax.numpy as jnp
from jax import lax
from jax.experimental import pallas as pl
from jax.experimental.pallas import tpu as pltpu
from jax.experimental.pallas import tpu_sc as plsc

N = 10000
E = 160000
NCORES = 2
NSUB = 16
NW = NCORES * NSUB
EPW = E // NW
ET = E // NSUB
EC = 125
NCH = ET // EC
DC = 100
NDC = EPW // DC
RB = 624
ZR = 48
P = 128

_mesh = plsc.VectorSubcoreMesh(core_axis_name="c", subcore_axis_name="s")
_sc_params = pltpu.CompilerParams(needs_layout_passes=False)


NDC2 = ET // DC


def _nrsqrt(x):
    xi = plsc.bitcast(x, jnp.int32)
    y = plsc.bitcast(jnp.full((16,), 0x5F3759DF, jnp.int32)
                     - lax.shift_right_logical(xi, 1), jnp.float32)
    hx = x * (-0.5)
    for _ in range(3):
        y = y * (hx * y * y + 1.5)
    return y


@functools.partial(
    pl.kernel, mesh=_mesh, compiler_params=_sc_params,
    out_type=[jax.ShapeDtypeStruct((NW, 1, EPW), jnp.float32),
              jax.ShapeDtypeStruct((1, N), jnp.float32)],
    scratch_types=[
        pltpu.VMEM((NDC2, DC), jnp.int32),
        pltpu.VMEM((NDC2, DC), jnp.float32),
        pltpu.VMEM((N,), jnp.float32),
        pltpu.VMEM((704,), jnp.float32),
        pltpu.VMEM((EPW + 16,), jnp.int32),
        pltpu.VMEM((EPW + 16,), jnp.int32),
        pltpu.VMEM((EPW + 16,), jnp.float32),
        pltpu.VMEM((EPW + 16,), jnp.float32),
        pltpu.VMEM((N,), jnp.float32),
        pltpu.VMEM_SHARED((N,), jnp.float32),
        pltpu.VMEM_SHARED((N,), jnp.float32),
        pltpu.SemaphoreType.DMA,
    ],
)
def _prep_kernel(dstd_hbm, wd_hbm, src_hbm, dst_hbm, w_hbm,
                 norm_hbm, dinv2_hbm,
                 dstv, wdv, dinvv, dslice, sv, dv, wv, nv, d2buf,
                 dacc, dtab, sem):
    c = lax.axis_index("c")
    s = lax.axis_index("s")
    wid = c * NSUB + s
    rows = jnp.where(s == NSUB - 1, RB + 16, RB)
    base = s * RB

    def z(i, carry):
        dinvv[pl.ds(i * 16, 16)] = jnp.zeros((16,), jnp.float32)
        return carry

    lax.fori_loop(0, N // 16, z, 0)
    pltpu.sync_copy(dstd_hbm.at[s], dstv)
    pltpu.sync_copy(wd_hbm.at[s], wdv)

    @pl.when(s == 0)
    def _():
        pltpu.sync_copy(dinvv, dacc)

    plsc.subcore_barrier()

    def cb(j, carry):
        pltpu.sync_copy(wdv.at[j], dacc.at[dstv.at[j]], add=True)
        return carry

    lax.fori_loop(0, NDC2, cb, 0)
    plsc.subcore_barrier()

    pltpu.sync_copy(dacc.at[pl.ds(base, RB)], dslice.at[pl.ds(0, RB)])

    @pl.when(s == NSUB - 1)
    def _():
        pltpu.sync_copy(dacc.at[pl.ds(NSUB * RB, 16)],
                        dslice.at[pl.ds(RB, 16)])

    def nb_(i, carry):
        deg = dslice[pl.ds(i * 16, 16)] + 1.0
        dslice[pl.ds(i * 16, 16)] = _nrsqrt(deg)
        return carry

    lax.fori_loop(0, 704 // 16, nb_, 0)
    pltpu.sync_copy(dslice.at[pl.ds(0, RB)], dtab.at[pl.ds(base, RB)])

    @pl.when(s == NSUB - 1)
    def _():
        pltpu.sync_copy(dslice.at[pl.ds(RB, 16)],
                        dtab.at[pl.ds(NSUB * RB, 16)])

    plsc.subcore_barrier()
    pltpu.sync_copy(dtab, dinvv)

    @pl.when((c == 0) & (s == 0))
    def _():
        def d2(i, carry):
            di = dinvv[pl.ds(i * 16, 16)]
            d2buf[pl.ds(i * 16, 16)] = di * di
            return carry

        lax.fori_loop(0, N // 16, d2, 0)
        pltpu.sync_copy(d2buf, dinv2_hbm.at[0])
    pltpu.sync_copy(src_hbm.at[wid, 0], sv.at[pl.ds(0, EPW)])
    pltpu.sync_copy(dst_hbm.at[wid, 0], dv.at[pl.ds(0, EPW)])
    pltpu.sync_copy(w_hbm.at[wid, 0], wv.at[pl.ds(0, EPW)])

    nit = pl.cdiv(EPW, 16)

    def eb(j, carry):
        si = jnp.clip(sv[pl.ds(j * 16, 16)], 0, N - 1)
        di = jnp.clip(dv[pl.ds(j * 16, 16)], 0, N - 1)
        gs = plsc.load_gather(dinvv, [si])
        gd = plsc.load_gather(dinvv, [di])
        nv[pl.ds(j * 16, 16)] = gs * wv[pl.ds(j * 16, 16)] * gd
        return carry

    lax.fori_loop(0, nit, eb, 0)
    pltpu.sync_copy(nv.at[pl.ds(0, EPW)], norm_hbm.at[wid, 0])


NB = 3
EB = 3
UNR = 5


def _make_agg(npan):
    ppc = npan // NCORES

    @functools.partial(
        pl.kernel, mesh=_mesh, compiler_params=_sc_params,
        out_type=[jax.ShapeDtypeStruct((N, P), jnp.float32)] * npan,
        scratch_types=(
            [pltpu.VMEM((3, EC), jnp.int32) for _ in range(EB)] +
            [pltpu.VMEM((EC, P), jnp.float32) for _ in range(NB)] +
            [pltpu.VMEM_SHARED((N, P), jnp.float32)] +
            [pltpu.SemaphoreType.DMA for _ in range(2 * NB + EB)]
        ),
    )
    def agg(*refs):
        ys = refs[:npan]
        edata_hbm = refs[npan]
        outs = refs[npan + 1:npan + 1 + npan]
        scr = refs[npan + 1 + npan:]
        ebufs = scr[0:EB]
        rbufs = scr[EB:EB + NB]
        acc = scr[EB + NB]
        sgs = scr[EB + NB + 1:EB + NB + 1 + NB]
        sss = scr[EB + NB + 1 + NB:EB + NB + 1 + 2 * NB]
        ses = scr[EB + NB + 1 + 2 * NB:]

        c = lax.axis_index("c")
        s = lax.axis_index("s")

        def scale(rows, ebuf):
            two = jnp.full((16,), 2, jnp.int32)

            @plsc.parallel_loop(0, EC, 1, unroll=UNR)
            def _(e):
                nbi = plsc.load_gather(
                    ebuf, [two, jnp.full((16,), e, jnp.int32)])
                nb = plsc.bitcast(nbi, jnp.float32)
                for k in range(P // 16):
                    rows[e, pl.ds(k * 16, 16)] = (
                        rows[e, pl.ds(k * 16, 16)] * nb)

        def run_panel(yref, oref):
            z0 = rbufs[0]

            def zrows(i, carry):
                for k in range(P // 16):
                    z0[i, pl.ds(k * 16, 16)] = jnp.zeros((16,), jnp.float32)
                return carry

            lax.fori_loop(0, EC, zrows, 0)
            for k in range(RB // 96):
                pltpu.sync_copy(z0.at[pl.ds(0, 96)],
                                acc.at[pl.ds(s * RB + k * 96, 96)])
            pltpu.sync_copy(z0.at[pl.ds(0, 48)],
                            acc.at[pl.ds(s * RB + 576, 48)])

            @pl.when(s == NSUB - 1)
            def _():
                pltpu.sync_copy(z0.at[pl.ds(0, 16)],
                                acc.at[pl.ds(NSUB * RB, 16)])

            plsc.subcore_barrier()

            pltpu.sync_copy(edata_hbm.at[s, 0], ebufs[0])
            pltpu.async_copy(yref.at[ebufs[0].at[0]], rbufs[0], sgs[0])
            pltpu.async_copy(edata_hbm.at[s, 1], ebufs[1], ses[1])

            def maybe_when(cond, fn):
                if isinstance(cond, bool):
                    if cond:
                        fn()
                else:
                    pl.when(cond)(fn)

            def step(ch, b):
                nb_ = (b + 1) % NB
                pb = (b + NB - 1) % NB

                def _gather():
                    pltpu.make_async_copy(
                        edata_hbm.at[s, ch + 1], ebufs[nb_], ses[nb_]).wait()
                    pltpu.async_copy(yref.at[ebufs[nb_].at[0]],
                                     rbufs[nb_], sgs[nb_])

                maybe_when(ch + 1 < NCH, _gather)

                pltpu.make_async_copy(
                    yref.at[ebufs[b].at[0]], rbufs[b], sgs[b]).wait()
                scale(rbufs[b], ebufs[b])

                def _drain():
                    pltpu.make_async_copy(
                        rbufs[pb], acc.at[ebufs[pb].at[1]], sss[pb]).wait()

                maybe_when(ch >= 1, _drain)

                def _pref():
                    pltpu.async_copy(edata_hbm.at[s, ch + 2],
                                     ebufs[pb], ses[pb])

                maybe_when(ch + 2 < NCH, _pref)

                pltpu.async_copy(rbufs[b], acc.at[ebufs[b].at[1]], sss[b],
                                 add=True)

            NMAIN = (NCH // NB) * NB

            def pack_body(g, carry):
                for u in range(NB):
                    step(g * NB + u, u)
                return carry

            lax.fori_loop(0, NCH // NB, pack_body, 0)
            for ch in range(NMAIN, NCH):
                step(ch, ch % NB)
            pltpu.make_async_copy(
                rbufs[(NCH - 1) % NB],
                acc.at[ebufs[(NCH - 1) % NB].at[1]],
                sss[(NCH - 1) % NB]).wait()

            plsc.subcore_barrier()
            pltpu.sync_copy(acc.at[pl.ds(s * RB, RB)],
                            oref.at[pl.ds(s * RB, RB)])

            @pl.when(s == NSUB - 1)
            def _():
                pltpu.sync_copy(acc.at[pl.ds(NSUB * RB, 16)],
                                oref.at[pl.ds(NSUB * RB, 16)])

            plsc.subcore_barrier()

        for cid in range(NCORES):
            @pl.when(c == cid)
            def _(cid=cid):
                for p in range(ppc):
                    pan = cid * ppc + p
                    run_panel(ys[pan], outs[pan])

    return agg


_agg4 = _make_agg(4)
_agg2 = _make_agg(2)


_BM = 256


def _mm(x, W, npan):
    K = x.shape[1]
    F = W.shape[1]

    def body(x_ref, w_ref, *o_refs):
        y = jnp.dot(x_ref[...], w_ref[...],
                    preferred_element_type=jnp.float32)
        for i, o in enumerate(o_refs):
            o[...] = y[:, i * P:(i + 1) * P]

    return pl.pallas_call(
        body,
        grid=(pl.cdiv(N, _BM),),
        in_specs=[pl.BlockSpec((_BM, K), lambda i: (i, 0)),
                  pl.BlockSpec((K, F), lambda i: (0, 0))],
        out_specs=[pl.BlockSpec((_BM, P), lambda i: (i, 0))] * npan,
        out_shape=[jax.ShapeDtypeStruct((N, P), jnp.float32)] * npan,
    )(x, W)


def _fuse(aggs, xws, dinv2c, b2d, W=None, npan_out=0, emit_h=False):
    npan_in = len(aggs)
    F = npan_in * P
    have_w = W is not None

    def body(*refs):
        k = 0
        aggr = refs[k:k + npan_in]; k += npan_in
        xwr = refs[k:k + npan_in]; k += npan_in
        d2 = refs[k]; br = refs[k + 1]; k += 2
        wr = None
        if have_w:
            wr = refs[k]; k += 1
        outs = refs[k:]
        agg = jnp.concatenate([r[...] for r in aggr], axis=1)
        xw = jnp.concatenate([r[...] for r in xwr], axis=1)
        h = jnp.maximum(agg + d2[...] * xw + br[...], 0.0)
        oi = 0
        if emit_h:
            outs[oi][...] = h
            oi += 1
        if have_w:
            y = jnp.dot(h, wr[...], preferred_element_type=jnp.float32)
            for i in range(npan_out):
                outs[oi + i][...] = y[:, i * P:(i + 1) * P]

    in_specs = ([pl.BlockSpec((_BM, P), lambda i: (i, 0))] * (2 * npan_in) +
                [pl.BlockSpec((_BM, 1), lambda i: (i, 0)),
                 pl.BlockSpec((1, F), lambda i: (0, 0))])
    args = list(aggs) + list(xws) + [dinv2c, b2d]
    if have_w:
        in_specs.append(pl.BlockSpec(W.shape, lambda i: (0, 0)))
        args.append(W)
    out_shape, out_specs = [], []
    if emit_h:
        out_shape.append(jax.ShapeDtypeStruct((N, F), jnp.float32))
        out_specs.append(pl.BlockSpec((_BM, F), lambda i: (i, 0)))
    out_shape += [jax.ShapeDtypeStruct((N, P), jnp.float32)] * npan_out
    out_specs += [pl.BlockSpec((_BM, P), lambda i: (i, 0))] * npan_out

    res = pl.pallas_call(
        body,
        grid=(pl.cdiv(N, _BM),),
        in_specs=in_specs,
        out_specs=out_specs,
        out_shape=out_shape,
    )(*args)
    return res


def kernel(x, edge_index, edge_weight, W1, b1, W2, b2, W3, b3, W4, b4):
    src = edge_index[0]
    dst = edge_index[1]

    norm, dinv2_row = _prep_kernel(
        dst.reshape(NSUB, NDC2, DC), edge_weight.reshape(NSUB, NDC2, DC),
        src.reshape(NW, 1, EPW), dst.reshape(NW, 1, EPW),
        edge_weight.reshape(NW, 1, EPW))
    dinv2c = dinv2_row.reshape(N, 1)
    norm_i = lax.bitcast_convert_type(norm.reshape(NSUB, NCH, EC), jnp.int32)
    edata = jnp.stack([src.reshape(NSUB, NCH, EC),
                       dst.reshape(NSUB, NCH, EC), norm_i], axis=2)

    xw1 = _mm(x, W1, 4)
    agg1 = _agg4(*xw1, edata)
    xw2 = _fuse(agg1, xw1, dinv2c, b1.reshape(1, -1), W=W2, npan_out=2)
    agg2 = _agg2(*xw2, edata)
    xw3 = _fuse(agg2, xw2, dinv2c, b2.reshape(1, -1), W=W3, npan_out=4)
    agg3 = _agg4(*xw3, edata)
    res3 = _fuse(agg3, xw3, dinv2c, b3.reshape(1, -1), W=W4, npan_out=2,
                 emit_h=True)
    x_emb = res3[0]
    xw4 = res3[1:]
    agg4 = _agg2(*xw4, edata)
    out = _fuse(agg4, xw4, dinv2c, b4.reshape(1, -1), emit_h=True)[0]
    return (out, x_emb)
